# baseline (speedup 1.0000x reference)
import jax
import jax.numpy as jnp
from jax.experimental import pallas as pl
from jax.experimental.pallas import tpu as pltpu

_TRANSFORMS = [jnp.tanh, jax.nn.relu, jax.nn.gelu, jax.nn.sigmoid]

_I0, _I1 = 1, 2

_CHUNK_ROWS = 16
_DEPTH = 4
_SPLIT = 96


def _transform(v):
    return _TRANSFORMS[_I1](_TRANSFORMS[_I0](v))


def _make_body(row_off, nrows):
    n = nrows // _CHUNK_ROWS

    def body(x_hbm, o_hbm, in_buf, out_buf, in_sems, out_sems):
        def in_copy(c):
            s = c % _DEPTH
            return pltpu.make_async_copy(
                x_hbm.at[pl.ds(row_off + c * _CHUNK_ROWS, _CHUNK_ROWS), :],
                in_buf.at[s],
                in_sems.at[s],
            )

        def out_copy(c):
            s = c % _DEPTH
            return pltpu.make_async_copy(
                out_buf.at[s],
                o_hbm.at[pl.ds(c * _CHUNK_ROWS, _CHUNK_ROWS), :],
                out_sems.at[s],
            )

        for c in range(min(_DEPTH, n)):
            in_copy(c).start()
        for c in range(n):
            s = c % _DEPTH
            in_copy(c).wait()
            if c >= _DEPTH:
                out_copy(c - _DEPTH).wait()
            out_buf[s] = _transform(in_buf[s])
            out_copy(c).start()
            if c + _DEPTH < n:
                in_copy(c + _DEPTH).start()
        for c in range(max(n - _DEPTH, 0), n):
            out_copy(c).wait()

    return body


def _stream_call(x, row_off, nrows):
    return pl.pallas_call(
        _make_body(row_off, nrows),
        in_specs=[pl.BlockSpec(memory_space=pl.ANY)],
        out_specs=pl.BlockSpec(memory_space=pl.ANY),
        out_shape=jax.ShapeDtypeStruct((nrows, x.shape[1]), x.dtype),
        scratch_shapes=[
            pltpu.VMEM((_DEPTH, _CHUNK_ROWS, x.shape[1]), x.dtype),
            pltpu.VMEM((_DEPTH, _CHUNK_ROWS, x.shape[1]), x.dtype),
            pltpu.SemaphoreType.DMA((_DEPTH,)),
            pltpu.SemaphoreType.DMA((_DEPTH,)),
        ],
    )(x)


def kernel(x):
    rows, _ = x.shape
    out_a = _stream_call(x, 0, _SPLIT)
    out_b = _stream_call(x, _SPLIT, rows - _SPLIT)
    return jnp.concatenate([out_a, out_b], axis=0)


# speedup vs baseline: 2.0215x; 2.0215x over previous
import jax
import jax.numpy as jnp
from jax.experimental import pallas as pl
from jax.experimental.pallas import tpu as pltpu

_TRANSFORMS = [jnp.tanh, jax.nn.relu, jax.nn.gelu, jax.nn.sigmoid]

_I0, _I1 = 1, 2

_CHUNK_ROWS = 64
_DEPTH = 3


def _transform(v):
    return _TRANSFORMS[_I1](_TRANSFORMS[_I0](v))


def _body(x_hbm, o_hbm, in_buf, out_buf, in_sems, out_sems):
    rows = x_hbm.shape[0]
    n = rows // _CHUNK_ROWS

    def in_copy(c):
        s = c % _DEPTH
        return pltpu.make_async_copy(
            x_hbm.at[pl.ds(c * _CHUNK_ROWS, _CHUNK_ROWS), :],
            in_buf.at[s],
            in_sems.at[s],
        )

    def out_copy(c):
        s = c % _DEPTH
        return pltpu.make_async_copy(
            out_buf.at[s],
            o_hbm.at[pl.ds(c * _CHUNK_ROWS, _CHUNK_ROWS), :],
            out_sems.at[s],
        )

    for c in range(min(_DEPTH, n)):
        in_copy(c).start()
    for c in range(n):
        s = c % _DEPTH
        in_copy(c).wait()
        if c >= _DEPTH:
            out_copy(c - _DEPTH).wait()
        out_buf[s] = _transform(in_buf[s])
        out_copy(c).start()
        if c + _DEPTH < n:
            in_copy(c + _DEPTH).start()
    for c in range(max(n - _DEPTH, 0), n):
        out_copy(c).wait()


def kernel(x):
    return pl.pallas_call(
        _body,
        in_specs=[pl.BlockSpec(memory_space=pl.ANY)],
        out_specs=pl.BlockSpec(memory_space=pl.ANY),
        out_shape=jax.ShapeDtypeStruct(x.shape, x.dtype),
        scratch_shapes=[
            pltpu.VMEM((_DEPTH, _CHUNK_ROWS, x.shape[1]), x.dtype),
            pltpu.VMEM((_DEPTH, _CHUNK_ROWS, x.shape[1]), x.dtype),
            pltpu.SemaphoreType.DMA((_DEPTH,)),
            pltpu.SemaphoreType.DMA((_DEPTH,)),
        ],
    )(x)


# speedup vs baseline: 2.3496x; 1.1623x over previous
import jax
import jax.numpy as jnp
from jax.experimental import pallas as pl
from jax.experimental.pallas import tpu as pltpu

_TRANSFORMS = [jnp.tanh, jax.nn.relu, jax.nn.gelu, jax.nn.sigmoid]

_I0, _I1 = 1, 2

_CHUNKS = (8, 16, 32, 32, 32, 8)


def _transform(v):
    return _TRANSFORMS[_I1](_TRANSFORMS[_I0](v))


def _body(x_hbm, o_hbm, *refs):
    n = len(_CHUNKS)
    in_bufs = refs[:n]
    out_bufs = refs[n:2 * n]
    in_sems, out_sems = refs[2 * n], refs[2 * n + 1]
    offs = [sum(_CHUNKS[:i]) for i in range(n)]

    def in_copy(c):
        return pltpu.make_async_copy(
            x_hbm.at[pl.ds(offs[c], _CHUNKS[c]), :], in_bufs[c], in_sems.at[c]
        )

    def out_copy(c):
        return pltpu.make_async_copy(
            out_bufs[c], o_hbm.at[pl.ds(offs[c], _CHUNKS[c]), :], out_sems.at[c]
        )

    for c in range(n):
        in_copy(c).start()
    for c in range(n):
        in_copy(c).wait()
        out_bufs[c][...] = _transform(in_bufs[c][...])
        out_copy(c).start()
    for c in range(n):
        out_copy(c).wait()


def kernel(x):
    cols = x.shape[1]
    n = len(_CHUNKS)
    scratch = (
        [pltpu.VMEM((r, cols), x.dtype) for r in _CHUNKS]
        + [pltpu.VMEM((r, cols), x.dtype) for r in _CHUNKS]
        + [pltpu.SemaphoreType.DMA((n,)), pltpu.SemaphoreType.DMA((n,))]
    )
    return pl.pallas_call(
        _body,
        in_specs=[pl.BlockSpec(memory_space=pl.ANY)],
        out_specs=pl.BlockSpec(memory_space=pl.ANY),
        out_shape=jax.ShapeDtypeStruct(x.shape, x.dtype),
        scratch_shapes=scratch,
    )(x)
